# baseline (speedup 1.0000x reference)
import functools
import math

import jax
import jax.numpy as jnp
from jax import lax
from jax.experimental import pallas as pl
from jax.experimental.pallas import tpu as pltpu
from jax.experimental.pallas import tpu_sc as plsc

V = 100000
E = 128
B = 128
L = 32
NR = 100
N = B * L
NW = 32
TPW = N // NW
NSP = 112
CHUNK = 32
NCH = TPW // CHUNK

LOG_V = math.log(V)
LOG_NR = math.log(NR)
LOG_S = math.log(V * (V + 1) / 2.0)

_mesh = plsc.VectorSubcoreMesh(core_axis_name="c", subcore_axis_name="s")


@functools.partial(
    pl.kernel,
    mesh=_mesh,
    out_type=[
        jax.ShapeDtypeStruct((N // 128, 128), jnp.float32),
        jax.ShapeDtypeStruct((NSP, E), jnp.float32),
        jax.ShapeDtypeStruct((NSP,), jnp.float32),
    ],
    scratch_types=[
        pltpu.VMEM((TPW,), jnp.int32),
        pltpu.VMEM((TPW, E), jnp.float32),
        pltpu.VMEM((2, CHUNK, E), jnp.float32),
        pltpu.VMEM((TPW + 16,), jnp.float32),
        pltpu.VMEM((TPW + 16,), jnp.float32),
        pltpu.VMEM((NSP,), jnp.int32),
        pltpu.VMEM((NSP, E), jnp.float32),
        pltpu.VMEM((NSP,), jnp.float32),
        pltpu.VMEM_SHARED((V,), jnp.float32),
        pltpu.SemaphoreType.DMA,
        pltpu.SemaphoreType.DMA,
        pltpu.SemaphoreType.DMA,
        pltpu.SemaphoreType.DMA,
    ],
)
def _sc_gather(tgt_hbm, ns_hbm, inp_hbm, emb_hbm, bias_hbm,
               score_out, nb_out, nbias_out,
               idx_v, inp_v, rows2_v, bias_v, score_v,
               nidx_v, nrows_v, nbias_v, sbias,
               sem_r, sem_b, sem_i, nsem):
    wid = lax.axis_index("s") * 2 + lax.axis_index("c")
    base = wid * TPW

    cp_inp = pltpu.async_copy(inp_hbm.at[pl.ds(base, TPW)], inp_v, sem_i)
    pltpu.sync_copy(tgt_hbm.at[pl.ds(base, TPW)], idx_v)
    pltpu.async_copy(emb_hbm.at[idx_v.at[pl.ds(0, CHUNK)]],
                     rows2_v.at[0], sem_r)
    pltpu.async_copy(emb_hbm.at[idx_v.at[pl.ds(CHUNK, CHUNK)]],
                     rows2_v.at[1], sem_r)

    @pl.when(lax.axis_index("s") == 15)
    def _():
        pltpu.sync_copy(bias_hbm, sbias)

    @pl.when(wid == 0)
    def _():
        pltpu.sync_copy(ns_hbm, nidx_v)
        pltpu.async_copy(emb_hbm.at[nidx_v], nrows_v, nsem).wait()
        pltpu.sync_copy(nrows_v, nb_out)

    plsc.subcore_barrier()
    cp_bias = pltpu.async_copy(sbias.at[idx_v],
                               bias_v.at[pl.ds(0, TPW)], sem_b)

    @pl.when(wid == 1)
    def _():
        pltpu.sync_copy(ns_hbm, nidx_v)
        pltpu.async_copy(sbias.at[nidx_v], nbias_v, nsem).wait()
        pltpu.sync_copy(nbias_v, nbias_out)

    cp_inp.wait()
    cp_bias.wait()

    def chunk_body(c, _):
        buf = c & 1
        pltpu.make_async_copy(emb_hbm.at[pl.ds(0, CHUNK)],
                              rows2_v.at[buf], sem_r).wait()

        def body(k, _):
            i = c * CHUNK + k
            acc = rows2_v[buf, k, pl.ds(0, 16)] * inp_v[i, pl.ds(0, 16)]
            for j in range(1, E // 16):
                acc = acc + (rows2_v[buf, k, pl.ds(16 * j, 16)]
                             * inp_v[i, pl.ds(16 * j, 16)])
            total = acc[0]
            for lane in range(1, 16):
                total = total + acc[lane]
            total = total + bias_v[pl.ds(i, 16)][0]
            score_v[pl.ds(i, 16)] = jnp.full((16,), total, jnp.float32)
            return 0

        lax.fori_loop(0, CHUNK, body, 0)

        @pl.when(c + 2 < NCH)
        def _():
            nxt = (c + 2) * CHUNK
            pltpu.async_copy(emb_hbm.at[idx_v.at[pl.ds(nxt, CHUNK)]],
                             rows2_v.at[buf], sem_r)

        return 0

    lax.fori_loop(0, NCH, chunk_body, 0)
    pltpu.sync_copy(score_v.at[pl.ds(0, TPW)], score_out.at[wid])


BLK = 512
GR = BLK // 128


def _tc_loss_body(inp_ref, score_ref, tgt_ref, nb_ref, nbias_ref, ns_ref,
                  out_ref):
    step = pl.program_id(0)

    @pl.when(step == 0)
    def _():
        out_ref[0, 0] = 0.0

    tgt_f = tgt_ref[...].astype(jnp.float32)
    x_t = (score_ref[...] - LOG_V
           - (jnp.log(tgt_f + 1.0) - LOG_S) - LOG_NR)
    t_sum = jnp.sum(jnp.maximum(-x_t, 0.0)
                    + jnp.log(1.0 + jnp.exp(-jnp.abs(x_t))))

    ns_f = ns_ref[...].astype(jnp.float32)
    col_off = nbias_ref[...] - LOG_V - (jnp.log(ns_f + 1.0) - LOG_S) - LOG_NR
    scores = lax.dot_general(inp_ref[...], nb_ref[...],
                             (((1,), (1,)), ((), ())),
                             preferred_element_type=jnp.float32)
    x_n = scores + col_off
    sp = jnp.maximum(x_n, 0.0) + jnp.log(1.0 + jnp.exp(-jnp.abs(x_n)))
    mask = lax.broadcasted_iota(jnp.int32, (BLK, NSP), 1) < NR
    n_sum = jnp.sum(jnp.where(mask, sp, 0.0))

    out_ref[0, 0] += (t_sum + n_sum) * (1.0 / N)


_tc_loss = pl.pallas_call(
    _tc_loss_body,
    grid=(N // BLK,),
    in_specs=[
        pl.BlockSpec((BLK, E), lambda i: (i, 0)),
        pl.BlockSpec((1, GR, 128), lambda i: (i, 0, 0)),
        pl.BlockSpec((1, GR, 128), lambda i: (i, 0, 0)),
        pl.BlockSpec((NSP, E), lambda i: (0, 0)),
        pl.BlockSpec((1, NSP), lambda i: (0, 0)),
        pl.BlockSpec((1, NSP), lambda i: (0, 0)),
    ],
    out_specs=pl.BlockSpec(memory_space=pltpu.SMEM),
    out_shape=jax.ShapeDtypeStruct((1, 1), jnp.float32),
)


def kernel(target, inp, noise_samples, emb_weight, bias_weight):
    tgt = target.reshape(-1).astype(jnp.int32)
    ns = jnp.concatenate([noise_samples.astype(jnp.int32),
                          jnp.zeros((NSP - NR,), jnp.int32)])
    inp_flat = inp.reshape(N, E)
    bias1d = bias_weight.reshape(-1)
    score, nb, nbias = _sc_gather(tgt, ns, inp_flat, emb_weight, bias1d)
    out = _tc_loss(inp_flat, score.reshape(N // BLK, GR, 128),
                   tgt.reshape(N // BLK, GR, 128), nb,
                   nbias.reshape(1, NSP), ns.reshape(1, NSP))
    return out[0, 0]


# speedup vs baseline: 1.0823x; 1.0823x over previous
import functools
import math

import jax
import jax.numpy as jnp
from jax import lax
from jax.experimental import pallas as pl
from jax.experimental.pallas import tpu as pltpu
from jax.experimental.pallas import tpu_sc as plsc

V = 100000
E = 128
B = 128
L = 32
NR = 100
N = B * L
NW = 32
TPW = N // NW
NSP = 112
CHUNK = 32
NCH = TPW // CHUNK

LOG_V = math.log(V)
LOG_NR = math.log(NR)
LOG_S = math.log(V * (V + 1) / 2.0)

_mesh = plsc.VectorSubcoreMesh(core_axis_name="c", subcore_axis_name="s")


@functools.partial(
    pl.kernel,
    mesh=_mesh,
    out_type=[
        jax.ShapeDtypeStruct((N // 8, 128), jnp.float32),
        jax.ShapeDtypeStruct((NSP, E), jnp.float32),
        jax.ShapeDtypeStruct((NSP,), jnp.float32),
    ],
    scratch_types=[
        pltpu.VMEM((TPW,), jnp.int32),
        pltpu.VMEM((TPW, E), jnp.float32),
        pltpu.VMEM((2, CHUNK, E), jnp.float32),
        pltpu.VMEM((TPW + 16,), jnp.float32),
        pltpu.VMEM((TPW // 8, 128), jnp.float32),
        pltpu.VMEM((NSP,), jnp.int32),
        pltpu.VMEM((NSP, E), jnp.float32),
        pltpu.VMEM((NSP,), jnp.float32),
        pltpu.VMEM_SHARED((V,), jnp.float32),
        pltpu.SemaphoreType.DMA,
        pltpu.SemaphoreType.DMA,
        pltpu.SemaphoreType.DMA,
        pltpu.SemaphoreType.DMA,
    ],
)
def _sc_gather(tgt_hbm, ns_hbm, inp_hbm, emb_hbm, bias_hbm,
               score_out, nb_out, nbias_out,
               idx_v, inp_v, rows2_v, bias_v, score_v,
               nidx_v, nrows_v, nbias_v, sbias,
               sem_r, sem_b, sem_i, nsem):
    wid = lax.axis_index("s") * 2 + lax.axis_index("c")
    base = wid * TPW

    cp_inp = pltpu.async_copy(inp_hbm.at[pl.ds(base, TPW)], inp_v, sem_i)
    pltpu.sync_copy(tgt_hbm.at[pl.ds(base, TPW)], idx_v)
    pltpu.async_copy(emb_hbm.at[idx_v.at[pl.ds(0, CHUNK)]],
                     rows2_v.at[0], sem_r)
    pltpu.async_copy(emb_hbm.at[idx_v.at[pl.ds(CHUNK, CHUNK)]],
                     rows2_v.at[1], sem_r)

    @pl.when(lax.axis_index("s") == 15)
    def _():
        pltpu.sync_copy(bias_hbm, sbias)

    @pl.when(wid == 0)
    def _():
        pltpu.sync_copy(ns_hbm, nidx_v)
        pltpu.async_copy(emb_hbm.at[nidx_v], nrows_v, nsem).wait()
        pltpu.sync_copy(nrows_v, nb_out)

    plsc.subcore_barrier()
    cp_bias = pltpu.async_copy(sbias.at[idx_v],
                               bias_v.at[pl.ds(0, TPW)], sem_b)

    @pl.when(wid == 1)
    def _():
        pltpu.sync_copy(ns_hbm, nidx_v)
        pltpu.async_copy(sbias.at[nidx_v], nbias_v, nsem).wait()
        pltpu.sync_copy(nbias_v, nbias_out)

    cp_inp.wait()
    cp_bias.wait()
    lane0 = lax.iota(jnp.int32, 16) == 0

    def chunk_body(c, _):
        buf = c & 1
        pltpu.make_async_copy(emb_hbm.at[pl.ds(0, CHUNK)],
                              rows2_v.at[buf], sem_r).wait()

        def body(k, _):
            i = c * CHUNK + k
            acc = rows2_v[buf, k, pl.ds(0, 16)] * inp_v[i, pl.ds(0, 16)]
            for j in range(1, E // 16):
                acc = acc + (rows2_v[buf, k, pl.ds(16 * j, 16)]
                             * inp_v[i, pl.ds(16 * j, 16)])
            acc = acc + jnp.where(lane0, bias_v[pl.ds(i, 16)], 0.0)
            score_v[i >> 3, pl.ds((i & 7) * 16, 16)] = acc
            return 0

        lax.fori_loop(0, CHUNK, body, 0)

        @pl.when(c + 2 < NCH)
        def _():
            nxt = (c + 2) * CHUNK
            pltpu.async_copy(emb_hbm.at[idx_v.at[pl.ds(nxt, CHUNK)]],
                             rows2_v.at[buf], sem_r)

        return 0

    lax.fori_loop(0, NCH, chunk_body, 0)
    pltpu.sync_copy(score_v, score_out.at[pl.ds(wid * (TPW // 8), TPW // 8)])


BLK = 512
GR = BLK // 128


def _tc_loss_body(inp_ref, score_ref, tgt_ref, nb_ref, nbias_ref, ns_ref,
                  out_ref):
    step = pl.program_id(0)

    @pl.when(step == 0)
    def _():
        out_ref[0, 0] = 0.0

    lsel = (lax.broadcasted_iota(jnp.int32, (128, 8), 0) >> 4
            == lax.broadcasted_iota(jnp.int32, (128, 8), 1))
    msel = jnp.where(lsel, 1.0, 0.0)
    tdot = lax.dot_general(score_ref[...], msel, (((1,), (0,)), ((), ())),
                           preferred_element_type=jnp.float32)
    tgt_f = tgt_ref[...].astype(jnp.float32)
    x_t = (tdot - LOG_V - (jnp.log(tgt_f + 1.0) - LOG_S) - LOG_NR)
    t_sum = jnp.sum(jnp.maximum(-x_t, 0.0)
                    + jnp.log(1.0 + jnp.exp(-jnp.abs(x_t))))

    ns_f = ns_ref[...].astype(jnp.float32)
    col_off = nbias_ref[...] - LOG_V - (jnp.log(ns_f + 1.0) - LOG_S) - LOG_NR
    scores = lax.dot_general(inp_ref[...], nb_ref[...],
                             (((1,), (1,)), ((), ())),
                             preferred_element_type=jnp.float32)
    x_n = scores + col_off
    sp = jnp.maximum(x_n, 0.0) + jnp.log(1.0 + jnp.exp(-jnp.abs(x_n)))
    mask = lax.broadcasted_iota(jnp.int32, (BLK, NSP), 1) < NR
    n_sum = jnp.sum(jnp.where(mask, sp, 0.0))

    out_ref[0, 0] += (t_sum + n_sum) * (1.0 / N)


_tc_loss = pl.pallas_call(
    _tc_loss_body,
    grid=(N // BLK,),
    in_specs=[
        pl.BlockSpec((BLK, E), lambda i: (i, 0)),
        pl.BlockSpec((BLK // 8, 128), lambda i: (i, 0)),
        pl.BlockSpec((BLK // 8, 8), lambda i: (i, 0)),
        pl.BlockSpec((NSP, E), lambda i: (0, 0)),
        pl.BlockSpec((1, NSP), lambda i: (0, 0)),
        pl.BlockSpec((1, NSP), lambda i: (0, 0)),
    ],
    out_specs=pl.BlockSpec(memory_space=pltpu.SMEM),
    out_shape=jax.ShapeDtypeStruct((1, 1), jnp.float32),
)


def kernel(target, inp, noise_samples, emb_weight, bias_weight):
    tgt = target.reshape(-1).astype(jnp.int32)
    ns = jnp.concatenate([noise_samples.astype(jnp.int32),
                          jnp.zeros((NSP - NR,), jnp.int32)])
    inp_flat = inp.reshape(N, E)
    bias1d = bias_weight.reshape(-1)
    score, nb, nbias = _sc_gather(tgt, ns, inp_flat, emb_weight, bias1d)
    out = _tc_loss(inp_flat, score, tgt.reshape(N // 8, 8), nb,
                   nbias.reshape(1, NSP), ns.reshape(1, NSP))
    return out[0, 0]


# speedup vs baseline: 1.2247x; 1.1316x over previous
import functools
import math

import jax
import jax.numpy as jnp
from jax import lax
from jax.experimental import pallas as pl
from jax.experimental.pallas import tpu as pltpu
from jax.experimental.pallas import tpu_sc as plsc

V = 100000
E = 128
B = 128
L = 32
NR = 100
N = B * L
NW = 32
TPW = N // NW
NSP = 112
CHUNK = 32
NCH = TPW // CHUNK

LOG_V = math.log(V)
LOG_NR = math.log(NR)
LOG_S = math.log(V * (V + 1) / 2.0)

_mesh = plsc.VectorSubcoreMesh(core_axis_name="c", subcore_axis_name="s")


@functools.partial(
    pl.kernel,
    mesh=_mesh,
    out_type=[
        jax.ShapeDtypeStruct((N // 8, 128), jnp.float32),
        jax.ShapeDtypeStruct((NSP, E), jnp.float32),
        jax.ShapeDtypeStruct((NSP,), jnp.float32),
        jax.ShapeDtypeStruct((NSP,), jnp.int32),
    ],
    scratch_types=[
        pltpu.VMEM((TPW,), jnp.int32),
        pltpu.VMEM((TPW, E), jnp.float32),
        pltpu.VMEM((2, CHUNK, E), jnp.float32),
        pltpu.VMEM((TPW + 16,), jnp.float32),
        pltpu.VMEM((TPW // 8, 128), jnp.float32),
        pltpu.VMEM((16,), jnp.int32),
        pltpu.VMEM((16, E), jnp.float32),
        pltpu.VMEM((16,), jnp.float32),
        pltpu.VMEM_SHARED((V,), jnp.float32),
        pltpu.SemaphoreType.DMA,
        pltpu.SemaphoreType.DMA,
        pltpu.SemaphoreType.DMA,
        pltpu.SemaphoreType.DMA,
    ],
)
def _sc_gather(tgt_hbm, ns_hbm, inp_hbm, emb_hbm, bias_hbm,
               score_out, nb_out, nbias_out, nsp_out,
               idx_v, inp_v, rows2_v, bias_v, score_v,
               nidx16_v, nrows_v, nbias16_v, sbias,
               sem_r, sem_b, sem_i, nsem):
    wid = lax.axis_index("s") * 2 + lax.axis_index("c")
    base = wid * TPW

    @pl.when(lax.axis_index("s") == 15)
    def _():
        pltpu.sync_copy(bias_hbm, sbias)

    cp_inp = pltpu.async_copy(inp_hbm.at[pl.ds(base, TPW)], inp_v, sem_i)
    pltpu.sync_copy(tgt_hbm.at[pl.ds(base, TPW)], idx_v)
    pltpu.async_copy(emb_hbm.at[idx_v.at[pl.ds(0, CHUNK)]],
                     rows2_v.at[0], sem_r)
    pltpu.async_copy(emb_hbm.at[idx_v.at[pl.ds(CHUNK, CHUNK)]],
                     rows2_v.at[1], sem_r)

    lanes16 = lax.iota(jnp.int32, 16)
    is_noise_tile = wid < 7
    nbase = jnp.where(wid < 6, wid * 16, 88)

    @pl.when(is_noise_tile)
    def _():
        @pl.when(wid < 6)
        def _():
            pltpu.sync_copy(ns_hbm.at[pl.ds(nbase, 16)], nidx16_v)

        @pl.when(wid == 6)
        def _():
            pltpu.sync_copy(ns_hbm.at[pl.ds(88, 8)], nidx16_v.at[pl.ds(0, 8)])
            pltpu.sync_copy(ns_hbm.at[pl.ds(96, 4)], nidx16_v.at[pl.ds(8, 4)])
            got = nidx16_v[pl.ds(0, 16)]
            nidx16_v[pl.ds(0, 16)] = jnp.where(lanes16 < 12, got, 0)

        pltpu.sync_copy(nidx16_v, nsp_out.at[pl.ds(nbase, 16)])
        pltpu.async_copy(emb_hbm.at[nidx16_v], nrows_v, nsem).wait()
        pltpu.sync_copy(nrows_v, nb_out.at[pl.ds(nbase, 16)])

    plsc.subcore_barrier()
    cp_bias = pltpu.async_copy(sbias.at[idx_v],
                               bias_v.at[pl.ds(0, TPW)], sem_b)

    @pl.when(is_noise_tile)
    def _():
        nbase = wid * 16
        pltpu.async_copy(sbias.at[nidx16_v], nbias16_v, nsem).wait()
        pltpu.sync_copy(nbias16_v, nbias_out.at[pl.ds(nbase, 16)])

    cp_inp.wait()
    cp_bias.wait()
    lane0 = lax.iota(jnp.int32, 16) == 0

    def chunk_body(c, _):
        buf = c & 1
        pltpu.make_async_copy(emb_hbm.at[pl.ds(0, CHUNK)],
                              rows2_v.at[buf], sem_r).wait()

        def body(k, _):
            i = c * CHUNK + k
            acc = rows2_v[buf, k, pl.ds(0, 16)] * inp_v[i, pl.ds(0, 16)]
            for j in range(1, E // 16):
                acc = acc + (rows2_v[buf, k, pl.ds(16 * j, 16)]
                             * inp_v[i, pl.ds(16 * j, 16)])
            acc = acc + jnp.where(lane0, bias_v[pl.ds(i, 16)], 0.0)
            score_v[i >> 3, pl.ds((i & 7) * 16, 16)] = acc
            return 0

        lax.fori_loop(0, CHUNK, body, 0)

        @pl.when(c + 2 < NCH)
        def _():
            nxt = (c + 2) * CHUNK
            pltpu.async_copy(emb_hbm.at[idx_v.at[pl.ds(nxt, CHUNK)]],
                             rows2_v.at[buf], sem_r)

        return 0

    lax.fori_loop(0, NCH, chunk_body, 0)
    pltpu.sync_copy(score_v, score_out.at[pl.ds(wid * (TPW // 8), TPW // 8)])


BLK = 1024
GR = BLK // 128


def _tc_loss_body(inp_ref, score_ref, tgt_ref, nb_ref, nbias_ref, ns_ref,
                  out_ref):
    step = pl.program_id(0)

    @pl.when(step == 0)
    def _():
        out_ref[0, 0] = 0.0

    lsel = (lax.broadcasted_iota(jnp.int32, (128, 8), 0) >> 4
            == lax.broadcasted_iota(jnp.int32, (128, 8), 1))
    msel = jnp.where(lsel, 1.0, 0.0)
    tdot = lax.dot_general(score_ref[...], msel, (((1,), (0,)), ((), ())),
                           preferred_element_type=jnp.float32)
    tgt_f = tgt_ref[...].astype(jnp.float32)
    x_t = (tdot - LOG_V - (jnp.log(tgt_f + 1.0) - LOG_S) - LOG_NR)
    t_sum = jnp.sum(jnp.maximum(-x_t, 0.0)
                    + jnp.log(1.0 + jnp.exp(-jnp.abs(x_t))))

    ns_f = ns_ref[...].astype(jnp.float32)
    col_off = nbias_ref[...] - LOG_V - (jnp.log(ns_f + 1.0) - LOG_S) - LOG_NR
    scores = lax.dot_general(inp_ref[...], nb_ref[...],
                             (((1,), (1,)), ((), ())),
                             preferred_element_type=jnp.float32)
    x_n = scores + col_off
    sp = jnp.maximum(x_n, 0.0) + jnp.log(1.0 + jnp.exp(-jnp.abs(x_n)))
    mask = lax.broadcasted_iota(jnp.int32, (BLK, NSP), 1) < NR
    n_sum = jnp.sum(jnp.where(mask, sp, 0.0))

    out_ref[0, 0] += (t_sum + n_sum) * (1.0 / N)


_tc_loss = pl.pallas_call(
    _tc_loss_body,
    grid=(N // BLK,),
    in_specs=[
        pl.BlockSpec((BLK, E), lambda i: (i, 0)),
        pl.BlockSpec((BLK // 8, 128), lambda i: (i, 0)),
        pl.BlockSpec((BLK // 8, 8), lambda i: (i, 0)),
        pl.BlockSpec((NSP, E), lambda i: (0, 0)),
        pl.BlockSpec((1, NSP), lambda i: (0, 0)),
        pl.BlockSpec((1, NSP), lambda i: (0, 0)),
    ],
    out_specs=pl.BlockSpec(memory_space=pltpu.SMEM),
    out_shape=jax.ShapeDtypeStruct((1, 1), jnp.float32),
)


def kernel(target, inp, noise_samples, emb_weight, bias_weight):
    tgt = target.reshape(-1).astype(jnp.int32)
    ns = noise_samples.astype(jnp.int32)
    inp_flat = inp.reshape(N, E)
    bias1d = bias_weight.reshape(-1)
    score, nb, nbias, nsp = _sc_gather(tgt, ns, inp_flat, emb_weight, bias1d)
    out = _tc_loss(inp_flat, score, tgt.reshape(N // 8, 8), nb,
                   nbias.reshape(1, NSP), nsp.reshape(1, NSP))
    return out[0, 0]


# speedup vs baseline: 1.2652x; 1.0331x over previous
import functools
import math

import jax
import jax.numpy as jnp
from jax import lax
from jax.experimental import pallas as pl
from jax.experimental.pallas import tpu as pltpu
from jax.experimental.pallas import tpu_sc as plsc

V = 100000
E = 128
B = 128
L = 32
NR = 100
N = B * L
NW = 32
TPW = N // NW
NSP = 112
CHUNK = 32
NCH = TPW // CHUNK

LOG_V = math.log(V)
LOG_NR = math.log(NR)
LOG_S = math.log(V * (V + 1) / 2.0)

_mesh = plsc.VectorSubcoreMesh(core_axis_name="c", subcore_axis_name="s")


@functools.partial(
    pl.kernel,
    mesh=_mesh,
    out_type=[
        jax.ShapeDtypeStruct((N // 8, 128), jnp.float32),
        jax.ShapeDtypeStruct((NSP, E), jnp.float32),
        jax.ShapeDtypeStruct((NSP,), jnp.float32),
        jax.ShapeDtypeStruct((NSP,), jnp.int32),
    ],
    scratch_types=[
        pltpu.VMEM((TPW,), jnp.int32),
        pltpu.VMEM((TPW, E), jnp.float32),
        pltpu.VMEM((2, CHUNK, E), jnp.float32),
        pltpu.VMEM((TPW + 16,), jnp.float32),
        pltpu.VMEM((TPW // 8, 128), jnp.float32),
        pltpu.VMEM((16,), jnp.int32),
        pltpu.VMEM((16, E), jnp.float32),
        pltpu.VMEM((16,), jnp.float32),
        pltpu.VMEM_SHARED((V,), jnp.float32),
        pltpu.SemaphoreType.DMA,
        pltpu.SemaphoreType.DMA,
        pltpu.SemaphoreType.DMA,
        pltpu.SemaphoreType.DMA,
    ],
)
def _sc_gather(tgt_hbm, ns_hbm, inp_hbm, emb_hbm, bias_hbm,
               score_out, nb_out, nbias_out, nsp_out,
               idx_v, inp_v, rows2_v, bias_v, score_v,
               nidx16_v, nrows_v, nbias16_v, sbias,
               sem_r, sem_b, sem_i, nsem):
    wid = lax.axis_index("s") * 2 + lax.axis_index("c")
    base = wid * TPW

    @pl.when(lax.axis_index("s") == 15)
    def _():
        pltpu.sync_copy(bias_hbm, sbias)

    cp_inp = pltpu.async_copy(inp_hbm.at[pl.ds(base, TPW)], inp_v, sem_i)
    pltpu.sync_copy(tgt_hbm.at[pl.ds(base, TPW)], idx_v)
    pltpu.async_copy(emb_hbm.at[idx_v.at[pl.ds(0, CHUNK)]],
                     rows2_v.at[0], sem_r)
    pltpu.async_copy(emb_hbm.at[idx_v.at[pl.ds(CHUNK, CHUNK)]],
                     rows2_v.at[1], sem_r)

    lanes16 = lax.iota(jnp.int32, 16)
    is_noise_tile = wid < 7
    nbase = jnp.where(wid < 6, wid * 16, 88)

    @pl.when(is_noise_tile)
    def _():
        @pl.when(wid < 6)
        def _():
            pltpu.sync_copy(ns_hbm.at[pl.ds(nbase, 16)], nidx16_v)

        @pl.when(wid == 6)
        def _():
            pltpu.sync_copy(ns_hbm.at[pl.ds(88, 8)], nidx16_v.at[pl.ds(0, 8)])
            pltpu.sync_copy(ns_hbm.at[pl.ds(96, 4)], nidx16_v.at[pl.ds(8, 4)])
            got = nidx16_v[pl.ds(0, 16)]
            nidx16_v[pl.ds(0, 16)] = jnp.where(lanes16 < 12, got, 0)

        pltpu.sync_copy(nidx16_v, nsp_out.at[pl.ds(nbase, 16)])
        pltpu.async_copy(emb_hbm.at[nidx16_v], nrows_v, nsem).wait()
        pltpu.sync_copy(nrows_v, nb_out.at[pl.ds(nbase, 16)])

    plsc.subcore_barrier()
    cp_bias = pltpu.async_copy(sbias.at[idx_v],
                               bias_v.at[pl.ds(0, TPW)], sem_b)

    @pl.when(is_noise_tile)
    def _():
        nbase = wid * 16
        pltpu.async_copy(sbias.at[nidx16_v], nbias16_v, nsem).wait()
        pltpu.sync_copy(nbias16_v, nbias_out.at[pl.ds(nbase, 16)])

    cp_inp.wait()
    cp_bias.wait()
    lane0 = lax.iota(jnp.int32, 16) == 0

    def chunk_body(c, _):
        buf = c & 1
        pltpu.make_async_copy(emb_hbm.at[pl.ds(0, CHUNK)],
                              rows2_v.at[buf], sem_r).wait()

        def body(k, _):
            i = c * CHUNK + k
            acc = rows2_v[buf, k, pl.ds(0, 16)] * inp_v[i, pl.ds(0, 16)]
            for j in range(1, E // 16):
                acc = acc + (rows2_v[buf, k, pl.ds(16 * j, 16)]
                             * inp_v[i, pl.ds(16 * j, 16)])
            acc = acc + jnp.where(lane0, bias_v[pl.ds(i, 16)], 0.0)
            score_v[i >> 3, pl.ds((i & 7) * 16, 16)] = acc
            return 0

        lax.fori_loop(0, CHUNK, body, 0)

        @pl.when(c + 2 < NCH)
        def _():
            nxt = (c + 2) * CHUNK
            pltpu.async_copy(emb_hbm.at[idx_v.at[pl.ds(nxt, CHUNK)]],
                             rows2_v.at[buf], sem_r)

        return 0

    lax.fori_loop(0, NCH, chunk_body, 0)
    pltpu.sync_copy(score_v, score_out.at[pl.ds(wid * (TPW // 8), TPW // 8)])


BLK = 2048
GR = BLK // 128


def _tc_loss_body(inp_ref, score_ref, tgt_ref, nb_ref, nbias_ref, ns_ref,
                  out_ref):
    step = pl.program_id(0)

    @pl.when(step == 0)
    def _():
        out_ref[0, 0] = 0.0

    lsel = (lax.broadcasted_iota(jnp.int32, (128, 8), 0) >> 4
            == lax.broadcasted_iota(jnp.int32, (128, 8), 1))
    msel = jnp.where(lsel, 1.0, 0.0)
    tdot = lax.dot_general(score_ref[...], msel, (((1,), (0,)), ((), ())),
                           preferred_element_type=jnp.float32)
    tgt_f = tgt_ref[...].astype(jnp.float32)
    x_t = (tdot - LOG_V - (jnp.log(tgt_f + 1.0) - LOG_S) - LOG_NR)
    t_sum = jnp.sum(jnp.maximum(-x_t, 0.0)
                    + jnp.log(1.0 + jnp.exp(-jnp.abs(x_t))))

    ns_f = ns_ref[...].astype(jnp.float32)
    col_off = nbias_ref[...] - LOG_V - (jnp.log(ns_f + 1.0) - LOG_S) - LOG_NR
    scores = lax.dot_general(inp_ref[...], nb_ref[...],
                             (((1,), (1,)), ((), ())),
                             preferred_element_type=jnp.float32)
    x_n = scores + col_off
    sp = jnp.maximum(x_n, 0.0) + jnp.log(1.0 + jnp.exp(-jnp.abs(x_n)))
    mask = lax.broadcasted_iota(jnp.int32, (BLK, NSP), 1) < NR
    n_sum = jnp.sum(jnp.where(mask, sp, 0.0))

    out_ref[0, 0] += (t_sum + n_sum) * (1.0 / N)


_tc_loss = pl.pallas_call(
    _tc_loss_body,
    grid=(N // BLK,),
    in_specs=[
        pl.BlockSpec((BLK, E), lambda i: (i, 0)),
        pl.BlockSpec((BLK // 8, 128), lambda i: (i, 0)),
        pl.BlockSpec((BLK // 8, 8), lambda i: (i, 0)),
        pl.BlockSpec((NSP, E), lambda i: (0, 0)),
        pl.BlockSpec((1, NSP), lambda i: (0, 0)),
        pl.BlockSpec((1, NSP), lambda i: (0, 0)),
    ],
    out_specs=pl.BlockSpec(memory_space=pltpu.SMEM),
    out_shape=jax.ShapeDtypeStruct((1, 1), jnp.float32),
)


def kernel(target, inp, noise_samples, emb_weight, bias_weight):
    tgt = target.reshape(-1).astype(jnp.int32)
    ns = noise_samples.astype(jnp.int32)
    inp_flat = inp.reshape(N, E)
    bias1d = bias_weight.reshape(-1)
    score, nb, nbias, nsp = _sc_gather(tgt, ns, inp_flat, emb_weight, bias1d)
    out = _tc_loss(inp_flat, score, tgt.reshape(N // 8, 8), nb,
                   nbias.reshape(1, NSP), nsp.reshape(1, NSP))
    return out[0, 0]


# speedup vs baseline: 1.2850x; 1.0156x over previous
import functools
import math

import jax
import jax.numpy as jnp
from jax import lax
from jax.experimental import pallas as pl
from jax.experimental.pallas import tpu as pltpu
from jax.experimental.pallas import tpu_sc as plsc

V = 100000
E = 128
B = 128
L = 32
NR = 100
N = B * L
NW = 32
TPW = N // NW
NSP = 112
CHUNK = 32
NCH = TPW // CHUNK

LOG_V = math.log(V)
LOG_NR = math.log(NR)
LOG_S = math.log(V * (V + 1) / 2.0)

_mesh = plsc.VectorSubcoreMesh(core_axis_name="c", subcore_axis_name="s")


@functools.partial(
    pl.kernel,
    mesh=_mesh,
    out_type=[
        jax.ShapeDtypeStruct((N // 8, 128), jnp.float32),
        jax.ShapeDtypeStruct((NSP, E), jnp.float32),
        jax.ShapeDtypeStruct((NSP,), jnp.float32),
        jax.ShapeDtypeStruct((NSP,), jnp.int32),
    ],
    scratch_types=[
        pltpu.VMEM((TPW,), jnp.int32),
        pltpu.VMEM((TPW, E), jnp.float32),
        pltpu.VMEM((2, CHUNK, E), jnp.float32),
        pltpu.VMEM((TPW + 16,), jnp.float32),
        pltpu.VMEM((TPW // 8, 128), jnp.float32),
        pltpu.VMEM((16,), jnp.int32),
        pltpu.VMEM((16, E), jnp.float32),
        pltpu.VMEM((16,), jnp.float32),
        pltpu.VMEM_SHARED((V,), jnp.float32),
        pltpu.SemaphoreType.DMA,
        pltpu.SemaphoreType.DMA,
        pltpu.SemaphoreType.DMA,
        pltpu.SemaphoreType.DMA,
    ],
)
def _sc_gather(tgt_hbm, ns_hbm, inp_hbm, emb_hbm, bias_hbm,
               score_out, nb_out, nbias_out, nsp_out,
               idx_v, inp_v, rows2_v, bias_v, score_v,
               nidx16_v, nrows_v, nbias16_v, sbias,
               sem_r, sem_b, sem_i, nsem):
    wid = lax.axis_index("s") * 2 + lax.axis_index("c")
    base = wid * TPW

    @pl.when(lax.axis_index("s") == 15)
    def _():
        pltpu.sync_copy(bias_hbm, sbias)

    cp_inp = pltpu.async_copy(inp_hbm.at[pl.ds(base, TPW)], inp_v, sem_i)
    pltpu.sync_copy(tgt_hbm.at[pl.ds(base, TPW)], idx_v)
    pltpu.async_copy(emb_hbm.at[idx_v.at[pl.ds(0, CHUNK)]],
                     rows2_v.at[0], sem_r)
    pltpu.async_copy(emb_hbm.at[idx_v.at[pl.ds(CHUNK, CHUNK)]],
                     rows2_v.at[1], sem_r)

    lanes16 = lax.iota(jnp.int32, 16)
    is_noise_tile = wid < 7
    nbase = jnp.where(wid < 6, wid * 16, 88)

    @pl.when(is_noise_tile)
    def _():
        @pl.when(wid < 6)
        def _():
            pltpu.sync_copy(ns_hbm.at[pl.ds(nbase, 16)], nidx16_v)

        @pl.when(wid == 6)
        def _():
            pltpu.sync_copy(ns_hbm.at[pl.ds(88, 8)], nidx16_v.at[pl.ds(0, 8)])
            pltpu.sync_copy(ns_hbm.at[pl.ds(96, 4)], nidx16_v.at[pl.ds(8, 4)])
            got = nidx16_v[pl.ds(0, 16)]
            nidx16_v[pl.ds(0, 16)] = jnp.where(lanes16 < 12, got, 0)

        pltpu.sync_copy(nidx16_v, nsp_out.at[pl.ds(nbase, 16)])
        pltpu.async_copy(emb_hbm.at[nidx16_v], nrows_v, nsem).wait()
        pltpu.sync_copy(nrows_v, nb_out.at[pl.ds(nbase, 16)])

    plsc.subcore_barrier()
    cp_bias = pltpu.async_copy(sbias.at[idx_v],
                               bias_v.at[pl.ds(0, TPW)], sem_b)

    @pl.when(is_noise_tile)
    def _():
        nbase = wid * 16
        pltpu.async_copy(sbias.at[nidx16_v], nbias16_v, nsem).wait()
        pltpu.sync_copy(nbias16_v, nbias_out.at[pl.ds(nbase, 16)])

    cp_inp.wait()
    cp_bias.wait()
    lane0 = lax.iota(jnp.int32, 16) == 0

    def chunk_body(c, _):
        buf = c & 1
        pltpu.make_async_copy(emb_hbm.at[pl.ds(0, CHUNK)],
                              rows2_v.at[buf], sem_r).wait()

        @plsc.parallel_loop(0, CHUNK, unroll=4)
        def body(k):
            i = c * CHUNK + k
            acc = rows2_v[buf, k, pl.ds(0, 16)] * inp_v[i, pl.ds(0, 16)]
            for j in range(1, E // 16):
                acc = acc + (rows2_v[buf, k, pl.ds(16 * j, 16)]
                             * inp_v[i, pl.ds(16 * j, 16)])
            acc = acc + jnp.where(lane0, bias_v[pl.ds(i, 16)], 0.0)
            score_v[i >> 3, pl.ds((i & 7) * 16, 16)] = acc

        @pl.when(c + 2 < NCH)
        def _():
            nxt = (c + 2) * CHUNK
            pltpu.async_copy(emb_hbm.at[idx_v.at[pl.ds(nxt, CHUNK)]],
                             rows2_v.at[buf], sem_r)

        return 0

    lax.fori_loop(0, NCH, chunk_body, 0)
    pltpu.sync_copy(score_v, score_out.at[pl.ds(wid * (TPW // 8), TPW // 8)])


BLK = 2048
GR = BLK // 128


def _tc_loss_body(inp_ref, score_ref, tgt_ref, nb_ref, nbias_ref, ns_ref,
                  out_ref):
    step = pl.program_id(0)

    @pl.when(step == 0)
    def _():
        out_ref[0, 0] = 0.0

    lsel = (lax.broadcasted_iota(jnp.int32, (128, 8), 0) >> 4
            == lax.broadcasted_iota(jnp.int32, (128, 8), 1))
    msel = jnp.where(lsel, 1.0, 0.0)
    tdot = lax.dot_general(score_ref[...], msel, (((1,), (0,)), ((), ())),
                           preferred_element_type=jnp.float32)
    tgt_f = tgt_ref[...].astype(jnp.float32)
    x_t = (tdot - LOG_V - (jnp.log(tgt_f + 1.0) - LOG_S) - LOG_NR)
    t_sum = jnp.sum(jnp.maximum(-x_t, 0.0)
                    + jnp.log(1.0 + jnp.exp(-jnp.abs(x_t))))

    ns_f = ns_ref[...].astype(jnp.float32)
    col_off = nbias_ref[...] - LOG_V - (jnp.log(ns_f + 1.0) - LOG_S) - LOG_NR
    scores = lax.dot_general(inp_ref[...], nb_ref[...],
                             (((1,), (1,)), ((), ())),
                             preferred_element_type=jnp.float32)
    x_n = scores + col_off
    sp = jnp.maximum(x_n, 0.0) + jnp.log(1.0 + jnp.exp(-jnp.abs(x_n)))
    mask = lax.broadcasted_iota(jnp.int32, (BLK, NSP), 1) < NR
    n_sum = jnp.sum(jnp.where(mask, sp, 0.0))

    out_ref[0, 0] += (t_sum + n_sum) * (1.0 / N)


_tc_loss = pl.pallas_call(
    _tc_loss_body,
    grid=(N // BLK,),
    in_specs=[
        pl.BlockSpec((BLK, E), lambda i: (i, 0)),
        pl.BlockSpec((BLK // 8, 128), lambda i: (i, 0)),
        pl.BlockSpec((BLK // 8, 8), lambda i: (i, 0)),
        pl.BlockSpec((NSP, E), lambda i: (0, 0)),
        pl.BlockSpec((1, NSP), lambda i: (0, 0)),
        pl.BlockSpec((1, NSP), lambda i: (0, 0)),
    ],
    out_specs=pl.BlockSpec(memory_space=pltpu.SMEM),
    out_shape=jax.ShapeDtypeStruct((1, 1), jnp.float32),
)


def kernel(target, inp, noise_samples, emb_weight, bias_weight):
    tgt = target.reshape(-1).astype(jnp.int32)
    ns = noise_samples.astype(jnp.int32)
    inp_flat = inp.reshape(N, E)
    bias1d = bias_weight.reshape(-1)
    score, nb, nbias, nsp = _sc_gather(tgt, ns, inp_flat, emb_weight, bias1d)
    out = _tc_loss(inp_flat, score, tgt.reshape(N // 8, 8), nb,
                   nbias.reshape(1, NSP), nsp.reshape(1, NSP))
    return out[0, 0]


# speedup vs baseline: 1.3824x; 1.0758x over previous
import functools
import math

import jax
import jax.numpy as jnp
from jax import lax
from jax.experimental import pallas as pl
from jax.experimental.pallas import tpu as pltpu
from jax.experimental.pallas import tpu_sc as plsc

V = 100000
E = 128
B = 128
L = 32
NR = 100
N = B * L
NW = 32
TPW = N // NW
NSP = 112
CHUNK = 32
NCH = TPW // CHUNK

LOG_V = math.log(V)
LOG_NR = math.log(NR)
LOG_S = math.log(V * (V + 1) / 2.0)

_mesh = plsc.VectorSubcoreMesh(core_axis_name="c", subcore_axis_name="s")


@functools.partial(
    pl.kernel,
    mesh=_mesh,
    out_type=[
        jax.ShapeDtypeStruct((N // 8, 128), jnp.float32),
        jax.ShapeDtypeStruct((NSP, E), jnp.float32),
        jax.ShapeDtypeStruct((NSP,), jnp.float32),
        jax.ShapeDtypeStruct((NSP,), jnp.int32),
    ],
    scratch_types=[
        pltpu.VMEM((TPW,), jnp.int32),
        pltpu.VMEM((TPW,), jnp.int32),
        pltpu.VMEM((TPW, E), jnp.float32),
        pltpu.VMEM((2, CHUNK, E), jnp.float32),
        pltpu.VMEM((TPW + 16,), jnp.float32),
        pltpu.VMEM((TPW // 8, 128), jnp.float32),
        pltpu.VMEM((16,), jnp.int32),
        pltpu.VMEM((16, E), jnp.float32),
        pltpu.VMEM((16,), jnp.float32),
        pltpu.VMEM_SHARED((V,), jnp.float32),
        pltpu.SemaphoreType.DMA,
        pltpu.SemaphoreType.DMA,
        pltpu.SemaphoreType.DMA,
        pltpu.SemaphoreType.DMA,
    ],
)
def _sc_gather(tgt_hbm, ns_hbm, inp_hbm, emb_hbm, bias_hbm,
               score_out, nb_out, nbias_out, nsp_out,
               idx_v, iidx_v, inp_v, rows2_v, bias_v, score_v,
               nidx16_v, nrows_v, nbias16_v, sbias,
               sem_r, sem_b, sem_i, nsem):
    wid = lax.axis_index("s") * 2 + lax.axis_index("c")
    base = wid * TPW

    @pl.when(lax.axis_index("s") == 15)
    def _():
        pltpu.sync_copy(bias_hbm, sbias)

    for g in range(TPW // 16):
        iidx_v[pl.ds(16 * g, 16)] = (lax.iota(jnp.int32, 16) + 16 * g) * L + wid
    cp_inp = pltpu.async_copy(inp_hbm.at[iidx_v], inp_v, sem_i)
    pltpu.sync_copy(tgt_hbm.at[pl.ds(base, TPW)], idx_v)
    pltpu.async_copy(emb_hbm.at[idx_v.at[pl.ds(0, CHUNK)]],
                     rows2_v.at[0], sem_r)
    pltpu.async_copy(emb_hbm.at[idx_v.at[pl.ds(CHUNK, CHUNK)]],
                     rows2_v.at[1], sem_r)

    lanes16 = lax.iota(jnp.int32, 16)
    is_noise_tile = wid < 7
    nbase = jnp.where(wid < 6, wid * 16, 88)

    @pl.when(is_noise_tile)
    def _():
        @pl.when(wid < 6)
        def _():
            pltpu.sync_copy(ns_hbm.at[pl.ds(nbase, 16)], nidx16_v)

        @pl.when(wid == 6)
        def _():
            pltpu.sync_copy(ns_hbm.at[pl.ds(88, 8)], nidx16_v.at[pl.ds(0, 8)])
            pltpu.sync_copy(ns_hbm.at[pl.ds(96, 4)], nidx16_v.at[pl.ds(8, 4)])
            got = nidx16_v[pl.ds(0, 16)]
            nidx16_v[pl.ds(0, 16)] = jnp.where(lanes16 < 12, got, 0)

        pltpu.sync_copy(nidx16_v, nsp_out.at[pl.ds(nbase, 16)])
        pltpu.async_copy(emb_hbm.at[nidx16_v], nrows_v, nsem).wait()
        pltpu.sync_copy(nrows_v, nb_out.at[pl.ds(nbase, 16)])

    plsc.subcore_barrier()
    cp_bias = pltpu.async_copy(sbias.at[idx_v],
                               bias_v.at[pl.ds(0, TPW)], sem_b)

    @pl.when(is_noise_tile)
    def _():
        nbase = wid * 16
        pltpu.async_copy(sbias.at[nidx16_v], nbias16_v, nsem).wait()
        pltpu.sync_copy(nbias16_v, nbias_out.at[pl.ds(nbase, 16)])

    cp_inp.wait()
    cp_bias.wait()
    lane0 = lax.iota(jnp.int32, 16) == 0

    def chunk_body(c, _):
        buf = c & 1
        pltpu.make_async_copy(emb_hbm.at[pl.ds(0, CHUNK)],
                              rows2_v.at[buf], sem_r).wait()

        @plsc.parallel_loop(0, CHUNK, unroll=4)
        def body(k):
            i = c * CHUNK + k
            acc = rows2_v[buf, k, pl.ds(0, 16)] * inp_v[i, pl.ds(0, 16)]
            for j in range(1, E // 16):
                acc = acc + (rows2_v[buf, k, pl.ds(16 * j, 16)]
                             * inp_v[i, pl.ds(16 * j, 16)])
            acc = acc + jnp.where(lane0, bias_v[pl.ds(i, 16)], 0.0)
            score_v[i >> 3, pl.ds((i & 7) * 16, 16)] = acc

        @pl.when(c + 2 < NCH)
        def _():
            nxt = (c + 2) * CHUNK
            pltpu.async_copy(emb_hbm.at[idx_v.at[pl.ds(nxt, CHUNK)]],
                             rows2_v.at[buf], sem_r)

        return 0

    lax.fori_loop(0, NCH, chunk_body, 0)
    pltpu.sync_copy(score_v, score_out.at[pl.ds(wid * (TPW // 8), TPW // 8)])


BLK = 2048
GR = BLK // 128


def _tc_loss_body(inp_ref, score_ref, tgt_ref, nb_ref, nbias_ref, ns_ref,
                  out_ref):
    step = pl.program_id(0)

    @pl.when(step == 0)
    def _():
        out_ref[0, 0] = 0.0

    lsel = (lax.broadcasted_iota(jnp.int32, (128, 8), 0) >> 4
            == lax.broadcasted_iota(jnp.int32, (128, 8), 1))
    msel = jnp.where(lsel, 1.0, 0.0)
    tdot = lax.dot_general(score_ref[...], msel, (((1,), (0,)), ((), ())),
                           preferred_element_type=jnp.float32)
    tgt_f = tgt_ref[...].astype(jnp.float32)
    x_t = (tdot - LOG_V - (jnp.log(tgt_f + 1.0) - LOG_S) - LOG_NR)
    t_sum = jnp.sum(jnp.maximum(-x_t, 0.0)
                    + jnp.log(1.0 + jnp.exp(-jnp.abs(x_t))))

    ns_f = ns_ref[...].astype(jnp.float32)
    col_off = nbias_ref[...] - LOG_V - (jnp.log(ns_f + 1.0) - LOG_S) - LOG_NR
    scores = lax.dot_general(inp_ref[...], nb_ref[...],
                             (((1,), (1,)), ((), ())),
                             preferred_element_type=jnp.float32)
    x_n = scores + col_off
    sp = jnp.maximum(x_n, 0.0) + jnp.log(1.0 + jnp.exp(-jnp.abs(x_n)))
    mask = lax.broadcasted_iota(jnp.int32, (BLK, NSP), 1) < NR
    n_sum = jnp.sum(jnp.where(mask, sp, 0.0))

    out_ref[0, 0] += (t_sum + n_sum) * (1.0 / N)


_tc_loss = pl.pallas_call(
    _tc_loss_body,
    grid=(N // BLK,),
    in_specs=[
        pl.BlockSpec((BLK, E), lambda i: (i, 0)),
        pl.BlockSpec((BLK // 8, 128), lambda i: (i, 0)),
        pl.BlockSpec((BLK // 8, 8), lambda i: (i, 0)),
        pl.BlockSpec((NSP, E), lambda i: (0, 0)),
        pl.BlockSpec((1, NSP), lambda i: (0, 0)),
        pl.BlockSpec((1, NSP), lambda i: (0, 0)),
    ],
    out_specs=pl.BlockSpec(memory_space=pltpu.SMEM),
    out_shape=jax.ShapeDtypeStruct((1, 1), jnp.float32),
)


def kernel(target, inp, noise_samples, emb_weight, bias_weight):
    tgt = target.T.reshape(-1).astype(jnp.int32)
    ns = noise_samples.astype(jnp.int32)
    inp_flat = inp.reshape(N, E)
    bias1d = bias_weight.T.reshape(-1)
    score, nb, nbias, nsp = _sc_gather(tgt, ns, inp_flat, emb_weight, bias1d)
    out = _tc_loss(inp_flat, score, tgt.reshape(N // 8, 8), nb,
                   nbias.reshape(1, NSP), nsp.reshape(1, NSP))
    return out[0, 0]
